# baseline (speedup 1.0000x reference)
import functools

import jax
import jax.numpy as jnp
from jax.experimental import pallas as pl
from jax.experimental.pallas import tpu as pltpu


def _round_up(x, m):
    return (x + m - 1) // m * m


def _fused_kernel(x_ref, w_ref, b_ref, o_ref, acc_ref, *, inv_hw):
    s = pl.program_id(1)
    cin_k = x_ref.shape[1]
    hw = x_ref.shape[2]

    x = x_ref[0]
    acc = x[:, 0:128]
    for j in range(1, hw // 128):
        acc = acc + x[:, j * 128:(j + 1) * 128]
    acc_ref[pl.ds(s * cin_k, cin_k)] = acc

    @pl.when(s == pl.num_programs(1) - 1)
    def _():
        m = jnp.dot(w_ref[...], acc_ref[...],
                    preferred_element_type=jnp.float32,
                    precision=jax.lax.Precision.DEFAULT)
        y = jnp.sum(m, axis=1, keepdims=True) * inv_hw + b_ref[...]
        o_ref[...] = jnp.broadcast_to(y[None], o_ref.shape)


def kernel(x_nchw, conv_w, conv_b):
    N, Cin, H, W = x_nchw.shape
    Cout = conv_w.shape[0]
    HW = H * W

    x = x_nchw.reshape(N, Cin, HW).astype(jnp.float32)
    w = conv_w.reshape(Cout, Cin).astype(jnp.float32)
    b = conv_b.reshape(Cout, 1).astype(jnp.float32)

    if HW % 128 != 0:
        x = jnp.pad(x, ((0, 0), (0, 0), (0, _round_up(HW, 128) - HW)))
    hwp = x.shape[2]

    cs = 2 if Cin % 16 == 0 else 1
    cin_k = Cin // cs

    vmem = int(min(56 << 20,
                   2 * cin_k * hwp * 4
                   + 2 * Cout * hwp * 4
                   + 2 * Cout * Cin * 4
                   + Cin * 128 * 4
                   + (6 << 20)))

    out = pl.pallas_call(
        functools.partial(_fused_kernel, inv_hw=1.0 / HW),
        out_shape=jax.ShapeDtypeStruct((N, Cout, hwp), jnp.float32),
        grid=(N, cs),
        in_specs=[
            pl.BlockSpec((1, cin_k, hwp), lambda n, s: (n, s, 0)),
            pl.BlockSpec((Cout, Cin), lambda n, s: (0, 0)),
            pl.BlockSpec((Cout, 1), lambda n, s: (0, 0)),
        ],
        out_specs=pl.BlockSpec((1, Cout, hwp), lambda n, s: (n, 0, 0)),
        scratch_shapes=[pltpu.VMEM((Cin, 128), jnp.float32)],
        compiler_params=pltpu.CompilerParams(
            dimension_semantics=("parallel", "arbitrary"),
            vmem_limit_bytes=vmem),
    )(x, w, b)

    if hwp != HW:
        out = out[:, :, :HW]
    return out.reshape(N, Cout, H, W)


# speedup vs baseline: 1.0433x; 1.0433x over previous
import functools

import jax
import jax.numpy as jnp
from jax.experimental import pallas as pl
from jax.experimental.pallas import tpu as pltpu


def _round_up(x, m):
    return (x + m - 1) // m * m


def _fused_kernel(x_ref, w_ref, b_ref, o_ref, *, inv_hw):
    nb = x_ref.shape[0]
    hw = x_ref.shape[2]
    for i in range(nb):
        x = x_ref[i]
        acc = x[:, 0:128]
        for j in range(1, hw // 128):
            acc = acc + x[:, j * 128:(j + 1) * 128]
        m = jnp.dot(w_ref[...], acc,
                    preferred_element_type=jnp.float32,
                    precision=jax.lax.Precision.DEFAULT)
        y = jnp.sum(m, axis=1, keepdims=True) * inv_hw + b_ref[...]
        o_ref[i] = jnp.broadcast_to(y, o_ref.shape[1:])


def kernel(x_nchw, conv_w, conv_b):
    N, Cin, H, W = x_nchw.shape
    Cout = conv_w.shape[0]
    HW = H * W

    x = x_nchw.reshape(N, Cin, HW).astype(jnp.float32)
    w = conv_w.reshape(Cout, Cin).astype(jnp.float32)
    b = conv_b.reshape(Cout, 1).astype(jnp.float32)

    if HW % 128 != 0:
        x = jnp.pad(x, ((0, 0), (0, 0), (0, _round_up(HW, 128) - HW)))
    hwp = x.shape[2]

    nb = 1
    for cand in (2, 1):
        if N % cand == 0 and 2 * cand * (Cin + Cout) * hwp * 4 < (40 << 20):
            nb = cand
            break

    vmem = int(min(56 << 20,
                   2 * nb * Cin * hwp * 4
                   + 2 * nb * Cout * hwp * 4
                   + 2 * Cout * Cin * 4
                   + (6 << 20)))

    out = pl.pallas_call(
        functools.partial(_fused_kernel, inv_hw=1.0 / HW),
        out_shape=jax.ShapeDtypeStruct((N, Cout, hwp), jnp.float32),
        grid=(N // nb,),
        in_specs=[
            pl.BlockSpec((nb, Cin, hwp), lambda n: (n, 0, 0)),
            pl.BlockSpec((Cout, Cin), lambda n: (0, 0)),
            pl.BlockSpec((Cout, 1), lambda n: (0, 0)),
        ],
        out_specs=pl.BlockSpec((nb, Cout, hwp), lambda n: (n, 0, 0)),
        compiler_params=pltpu.CompilerParams(
            dimension_semantics=("parallel",),
            vmem_limit_bytes=vmem),
    )(x, w, b)

    if hwp != HW:
        out = out[:, :, :HW]
    return out.reshape(N, Cout, H, W)
